# baseline (speedup 1.0000x reference)
import math

import jax
import jax.numpy as jnp
from jax import lax
from jax.experimental import pallas as pl
from jax.experimental.pallas import tpu as pltpu
from jax.experimental.pallas import tpu_sc as plsc

_K = 256
_T = 64
_S = 500
_A = 4
_D = 128

_NC = 2
_NS = 16
_NW = _NC * _NS
_PPW = _K // _NW
_PPW2 = _K // _NS


def _sample_body(g_ref, lw_ref, idx_ref, it0_ref, it1_ref):
    x = g_ref[...] + lw_ref[...][None, :]
    m = jnp.max(x, axis=1, keepdims=True)
    ii = lax.broadcasted_iota(jnp.int32, x.shape, 1)
    cand = jnp.where(x >= m, ii, jnp.int32(x.shape[1]))
    idx = jnp.min(cand, axis=1)
    idx_ref[...] = idx
    it0_ref[...] = idx * _T
    it1_ref[...] = idx * _T + 1


def _sample(gumbel_kk, log_weight_k):
    return pl.pallas_call(
        _sample_body,
        out_shape=(
            jax.ShapeDtypeStruct((_K,), jnp.int32),
            jax.ShapeDtypeStruct((_K,), jnp.int32),
            jax.ShapeDtypeStruct((_K,), jnp.int32),
        ),
    )(gumbel_kk, log_weight_k)


def _sc_gather_body(
    it0_hbm, it1_hbm, idx_hbm, lf_hbm, emb_hbm, lc_hbm, hs_hbm, lp_hbm,
    lf0_out, lf1_out, emb_out, lc_out, hs_out, lp_out,
    it0_v, it1_v, lf0_v, lf1_v, e0_v, e1_v, es_v,
    i16_v, j16_v, k16_v, a16_v, b16_v, o16_v, lp16_v,
    sem0, sem1,
):
    w = lax.axis_index("s") * _NC + lax.axis_index("c")
    base = w * _PPW

    pltpu.sync_copy(it0_hbm.at[pl.ds(base, _PPW)], it0_v)
    pltpu.sync_copy(it1_hbm.at[pl.ds(base, _PPW)], it1_v)

    cp0 = pltpu.async_copy(lf_hbm.at[it0_v], lf0_v, sem0)
    cp1 = pltpu.async_copy(lf_hbm.at[it1_v], lf1_v, sem1)
    cp0.wait()
    cp1.wait()
    pltpu.sync_copy(lf0_v, lf0_out.at[pl.ds(base, _PPW)])
    pltpu.sync_copy(lf1_v, lf1_out.at[pl.ds(base, _PPW)])

    cp0 = pltpu.async_copy(emb_hbm.at[it0_v], e0_v, sem0)
    cp1 = pltpu.async_copy(emb_hbm.at[it1_v], e1_v, sem1)
    cp0.wait()
    cp1.wait()
    for p in range(_PPW):
        for jc in range(_D // 16):
            sl = pl.ds(jc * 16, 16)
            es_v[p, sl] = (e0_v[p, sl] + e1_v[p, sl]) * 0.5
    pltpu.sync_copy(es_v, emb_out.at[pl.ds(base, _PPW)])

    @pl.when(w < _NS)
    def _scalars():
        b16 = w * _PPW2
        pltpu.sync_copy(it0_hbm.at[pl.ds(b16, _PPW2)], i16_v)
        pltpu.sync_copy(it1_hbm.at[pl.ds(b16, _PPW2)], j16_v)
        pltpu.sync_copy(idx_hbm.at[pl.ds(b16, _PPW2)], k16_v)
        pltpu.sync_copy(lc_hbm.at[i16_v], a16_v)
        pltpu.sync_copy(lc_hbm.at[j16_v], b16_v)
        o16_v[...] = a16_v[...] + b16_v[...]
        pltpu.sync_copy(o16_v, lc_out.at[pl.ds(b16, _PPW2)])
        pltpu.sync_copy(hs_hbm.at[i16_v], a16_v)
        pltpu.sync_copy(hs_hbm.at[j16_v], b16_v)
        o16_v[...] = a16_v[...] * jnp.int32(1000003) + b16_v[...]
        pltpu.sync_copy(o16_v, hs_out.at[pl.ds(b16, _PPW2)])
        pltpu.sync_copy(lp_hbm.at[k16_v], lp16_v)
        pltpu.sync_copy(lp16_v, lp_out.at[pl.ds(b16, _PPW2)])


def _sc_gather(it0, it1, idx, lf_flat, emb_flat, lc_flat, hs_flat, log_pi):
    mesh = plsc.VectorSubcoreMesh(
        core_axis_name="c", subcore_axis_name="s", num_cores=_NC,
        num_subcores=_NS)
    f = pl.kernel(
        _sc_gather_body,
        out_type=(
            jax.ShapeDtypeStruct((_K, _S * _A), jnp.float32),
            jax.ShapeDtypeStruct((_K, _S * _A), jnp.float32),
            jax.ShapeDtypeStruct((_K, _D), jnp.float32),
            jax.ShapeDtypeStruct((_K,), jnp.int32),
            jax.ShapeDtypeStruct((_K,), jnp.int32),
            jax.ShapeDtypeStruct((_K,), jnp.float32),
        ),
        mesh=mesh,
        compiler_params=pltpu.CompilerParams(use_tc_tiling_on_sc=False),
        scratch_types=[
            pltpu.VMEM((_PPW,), jnp.int32),
            pltpu.VMEM((_PPW,), jnp.int32),
            pltpu.VMEM((_PPW, _S * _A), jnp.float32),
            pltpu.VMEM((_PPW, _S * _A), jnp.float32),
            pltpu.VMEM((_PPW, _D), jnp.float32),
            pltpu.VMEM((_PPW, _D), jnp.float32),
            pltpu.VMEM((_PPW, _D), jnp.float32),
            pltpu.VMEM((_PPW2,), jnp.int32),
            pltpu.VMEM((_PPW2,), jnp.int32),
            pltpu.VMEM((_PPW2,), jnp.int32),
            pltpu.VMEM((_PPW2,), jnp.int32),
            pltpu.VMEM((_PPW2,), jnp.int32),
            pltpu.VMEM((_PPW2,), jnp.int32),
            pltpu.VMEM((_PPW2,), jnp.float32),
            pltpu.SemaphoreType.DMA,
            pltpu.SemaphoreType.DMA,
        ],
    )
    return f(it0, it1, idx, lf_flat, emb_flat, lc_flat, hs_flat, log_pi)


def _grp4(x):
    lane = lax.broadcasted_iota(jnp.int32, x.shape, 1)
    p2 = x + jnp.where(lane & 1 == 1, jnp.roll(x, 1, axis=1),
                       jnp.roll(x, -1, axis=1))
    return p2 + jnp.where(lane & 2 == 2, jnp.roll(p2, 2, axis=1),
                          jnp.roll(p2, -2, axis=1))


def _merge_body(lf0_ref, lf1_ref, lp_ref, lfn_ref, lw_ref):
    eb = jnp.exp(jnp.float32(-0.1))
    move = (1.0 - eb) / _A
    e1 = jnp.exp(lf0_ref[...])
    e2 = jnp.exp(lf1_ref[...])
    q1 = move * _grp4(e1) + eb * e1
    q2 = move * _grp4(e2) + eb * e2
    u = q1 * q2
    lfn_ref[...] = jnp.log(u)
    ll = 0.25 * jnp.sum(jnp.log(_grp4(u)), axis=1) - _S * math.log(_A)
    lw_ref[...] = (ll - lp_ref[0, 0, :])[None, None, :]


_BK = 128


def _merge(lf0, lf1, lp_g):
    grid = (_K // _BK,)
    lf_new2d, lw2d = pl.pallas_call(
        _merge_body,
        grid=grid,
        in_specs=[
            pl.BlockSpec((_BK, _S * _A), lambda i: (i, 0)),
            pl.BlockSpec((_BK, _S * _A), lambda i: (i, 0)),
            pl.BlockSpec((1, 1, _BK), lambda i: (i, 0, 0)),
        ],
        out_specs=(
            pl.BlockSpec((_BK, _S * _A), lambda i: (i, 0)),
            pl.BlockSpec((1, 1, _BK), lambda i: (i, 0, 0)),
        ),
        out_shape=(
            jax.ShapeDtypeStruct((_K, _S * _A), jnp.float32),
            jax.ShapeDtypeStruct((_K // _BK, 1, _BK), jnp.float32),
        ),
    )(lf0, lf1, lp_g.reshape(_K // _BK, 1, _BK))
    return lf_new2d, lw2d.reshape(_K)


def kernel(log_weight_K, log_pi_K, log_felsensteins_KxtxSxA, embeddings_KxtxD,
           leaf_counts_Kxt, hashes_Kxt):
    gumbel = jax.random.gumbel(jax.random.key(1), (_K, _K), jnp.float32)
    idx, it0, it1 = _sample(gumbel, log_weight_K)

    lf_flat = log_felsensteins_KxtxSxA.reshape(_K * _T, _S * _A)
    emb_flat = embeddings_KxtxD.reshape(_K * _T, _D)
    lc_flat = leaf_counts_Kxt.reshape(_K * _T)
    hs_flat = hashes_Kxt.reshape(_K * _T)

    lf0, lf1, emb_new, lc_new, hs_new, lp_g = _sc_gather(
        it0, it1, idx, lf_flat, emb_flat, lc_flat, hs_flat, log_pi_K)

    lf_new2d, lw_new = _merge(lf0, lf1, lp_g)
    return (lw_new, lf_new2d.reshape(_K, _S, _A), emb_new, lc_new, hs_new)


# speedup vs baseline: 3.1762x; 3.1762x over previous
import math

import jax
import jax.numpy as jnp
from jax import lax
from jax.experimental import pallas as pl
from jax.experimental.pallas import tpu as pltpu
from jax.experimental.pallas import tpu_sc as plsc

_K = 256
_T = 64
_S = 500
_A = 4
_D = 128
_SA = _S * _A
_SAP = 2048

_NC = 2
_NS = 16
_NW = _NC * _NS
_PPW = _K // _NW


def _sample_body(g_ref, lw_ref, sc_ref, it0_ref, it1_ref, lc_ref, hs_ref,
                 lp_ref):
    x = g_ref[...] + lw_ref[...][None, :]
    m = jnp.max(x, axis=1, keepdims=True)
    ii = lax.broadcasted_iota(jnp.int32, x.shape, 1)
    cand = jnp.where(x >= m, ii, jnp.int32(x.shape[1]))
    idx = jnp.min(cand, axis=1)
    it0_ref[...] = idx * 2
    it1_ref[...] = idx * 2 + 1
    oh = (idx[:, None] == ii).astype(jnp.float32)
    g = jnp.dot(oh, sc_ref[...], preferred_element_type=jnp.float32)
    lc_ref[...] = g[:, 0].astype(jnp.int32) + g[:, 1].astype(jnp.int32)
    hs_ref[...] = (g[:, 2].astype(jnp.int32) * jnp.int32(1000003)
                   + g[:, 3].astype(jnp.int32))
    lp_ref[...] = g[:, 4]


def _sample(gumbel_kk, log_weight_k, scalar_tbl):
    return pl.pallas_call(
        _sample_body,
        out_shape=(
            jax.ShapeDtypeStruct((_K,), jnp.int32),
            jax.ShapeDtypeStruct((_K,), jnp.int32),
            jax.ShapeDtypeStruct((_K,), jnp.int32),
            jax.ShapeDtypeStruct((_K,), jnp.int32),
            jax.ShapeDtypeStruct((_K,), jnp.float32),
        ),
    )(gumbel_kk, log_weight_k, scalar_tbl)


def _sc_gather_body(
    it0_hbm, it1_hbm, lf_hbm, emb_hbm,
    lf0_out, lf1_out, emb_out,
    it0_v, it1_v, lf0_v, lf1_v, e0_v, e1_v, es_v,
    sem0, sem1,
):
    w = lax.axis_index("s") * _NC + lax.axis_index("c")
    base = w * _PPW

    pltpu.sync_copy(it0_hbm.at[pl.ds(base, _PPW)], it0_v)
    pltpu.sync_copy(it1_hbm.at[pl.ds(base, _PPW)], it1_v)

    cp0 = pltpu.async_copy(lf_hbm.at[it0_v], lf0_v, sem0)
    cp1 = pltpu.async_copy(lf_hbm.at[it1_v], lf1_v, sem0)
    ce0 = pltpu.async_copy(emb_hbm.at[it0_v], e0_v, sem1)
    ce1 = pltpu.async_copy(emb_hbm.at[it1_v], e1_v, sem1)
    ce0.wait()
    ce1.wait()
    for p in range(_PPW):
        for jc in range(_D // 16):
            sl = pl.ds(jc * 16, 16)
            es_v[p, sl] = (e0_v[p, sl] + e1_v[p, sl]) * 0.5
    pltpu.sync_copy(es_v, emb_out.at[pl.ds(base, _PPW)])
    cp0.wait()
    cp1.wait()
    pltpu.sync_copy(lf0_v, lf0_out.at[pl.ds(base, _PPW)])
    pltpu.sync_copy(lf1_v, lf1_out.at[pl.ds(base, _PPW)])


def _sc_gather(it0, it1, lf01p, emb01):
    mesh = plsc.VectorSubcoreMesh(
        core_axis_name="c", subcore_axis_name="s", num_cores=_NC,
        num_subcores=_NS)
    f = pl.kernel(
        _sc_gather_body,
        out_type=(
            jax.ShapeDtypeStruct((_K, _SAP), jnp.float32),
            jax.ShapeDtypeStruct((_K, _SAP), jnp.float32),
            jax.ShapeDtypeStruct((_K, _D), jnp.float32),
        ),
        mesh=mesh,
        scratch_types=[
            pltpu.VMEM((_PPW,), jnp.int32),
            pltpu.VMEM((_PPW,), jnp.int32),
            pltpu.VMEM((_PPW, _SAP), jnp.float32),
            pltpu.VMEM((_PPW, _SAP), jnp.float32),
            pltpu.VMEM((_PPW, _D), jnp.float32),
            pltpu.VMEM((_PPW, _D), jnp.float32),
            pltpu.VMEM((_PPW, _D), jnp.float32),
            pltpu.SemaphoreType.DMA,
            pltpu.SemaphoreType.DMA,
        ],
    )
    return f(it0, it1, lf01p, emb01)


def _grp4(x):
    lane = lax.broadcasted_iota(jnp.int32, x.shape, 1)
    p2 = x + jnp.where(lane & 1 == 1, jnp.roll(x, 1, axis=1),
                       jnp.roll(x, -1, axis=1))
    return p2 + jnp.where(lane & 2 == 2, jnp.roll(p2, 2, axis=1),
                          jnp.roll(p2, -2, axis=1))


def _merge_body(lf0_ref, lf1_ref, lp_ref, lfn_ref, lw_ref):
    eb = jnp.exp(jnp.float32(-0.1))
    move = (1.0 - eb) / _A
    e1 = jnp.exp(lf0_ref[...])
    e2 = jnp.exp(lf1_ref[...])
    q1 = move * _grp4(e1) + eb * e1
    q2 = move * _grp4(e2) + eb * e2
    u = q1 * q2
    lfn_ref[...] = jnp.log(u)
    lane = lax.broadcasted_iota(jnp.int32, u.shape, 1)
    tu = jnp.where(lane < _SA, _grp4(u), 1.0)
    ll = 0.25 * jnp.sum(jnp.log(tu), axis=1) - _S * math.log(_A)
    lw_ref[...] = (ll - lp_ref[0, 0, :])[None, None, :]


_BK = 128


def _merge(lf0, lf1, lp_g):
    grid = (_K // _BK,)
    lf_new2d, lw2d = pl.pallas_call(
        _merge_body,
        grid=grid,
        in_specs=[
            pl.BlockSpec((_BK, _SAP), lambda i: (i, 0)),
            pl.BlockSpec((_BK, _SAP), lambda i: (i, 0)),
            pl.BlockSpec((1, 1, _BK), lambda i: (i, 0, 0)),
        ],
        out_specs=(
            pl.BlockSpec((_BK, _SAP), lambda i: (i, 0)),
            pl.BlockSpec((1, 1, _BK), lambda i: (i, 0, 0)),
        ),
        out_shape=(
            jax.ShapeDtypeStruct((_K, _SAP), jnp.float32),
            jax.ShapeDtypeStruct((_K // _BK, 1, _BK), jnp.float32),
        ),
    )(lf0, lf1, lp_g.reshape(_K // _BK, 1, _BK))
    return lf_new2d, lw2d.reshape(_K)


def kernel(log_weight_K, log_pi_K, log_felsensteins_KxtxSxA, embeddings_KxtxD,
           leaf_counts_Kxt, hashes_Kxt):
    gumbel = jax.random.gumbel(jax.random.key(1), (_K, _K), jnp.float32)

    lf01p = jnp.pad(
        log_felsensteins_KxtxSxA[:, :2].reshape(_K * 2, _SA),
        ((0, 0), (0, _SAP - _SA)))
    emb01 = embeddings_KxtxD[:, :2].reshape(_K * 2, _D)
    scalar_tbl = jnp.stack(
        [leaf_counts_Kxt[:, 0].astype(jnp.float32),
         leaf_counts_Kxt[:, 1].astype(jnp.float32),
         hashes_Kxt[:, 0].astype(jnp.float32),
         hashes_Kxt[:, 1].astype(jnp.float32),
         log_pi_K], axis=1)

    it0, it1, lc_new, hs_new, lp_g = _sample(gumbel, log_weight_K, scalar_tbl)
    lf0, lf1, emb_new = _sc_gather(it0, it1, lf01p, emb01)
    lf_new2d, lw_new = _merge(lf0, lf1, lp_g)
    lf_new = lf_new2d[:, :_SA].reshape(_K, _S, _A)
    return (lw_new, lf_new, emb_new, lc_new, hs_new)


# speedup vs baseline: 3.1763x; 1.0000x over previous
import math

import jax
import jax.numpy as jnp
from jax import lax
from jax.experimental import pallas as pl
from jax.experimental.pallas import tpu as pltpu
from jax.experimental.pallas import tpu_sc as plsc

_K = 256
_T = 64
_S = 500
_A = 4
_D = 128
_SA = _S * _A
_SAP = 2048

_NC = 2
_NS = 16
_NW = _NC * _NS
_PPW = _K // _NW


def _sample_body(g_ref, lw_ref, sc_ref, it0_ref, it1_ref, lc_ref, hs_ref,
                 lp_ref):
    x = g_ref[...] + lw_ref[...][None, :]
    m = jnp.max(x, axis=1, keepdims=True)
    ii = lax.broadcasted_iota(jnp.int32, x.shape, 1)
    cand = jnp.where(x >= m, ii, jnp.int32(x.shape[1]))
    idx = jnp.min(cand, axis=1)
    it0_ref[...] = idx * 2
    it1_ref[...] = idx * 2 + 1
    eq = idx[:, None] == ii
    t = sc_ref[...]

    def pick(row):
        return jnp.sum(jnp.where(eq, row[None, :], 0), axis=1)

    lc_ref[...] = pick(t[0]) + pick(t[1])
    hs_ref[...] = pick(t[2]) * jnp.int32(1000003) + pick(t[3])
    lp_ref[...] = lax.bitcast_convert_type(pick(t[4]), jnp.float32)


def _sample(gumbel_kk, log_weight_k, scalar_tbl):
    return pl.pallas_call(
        _sample_body,
        out_shape=(
            jax.ShapeDtypeStruct((_K,), jnp.int32),
            jax.ShapeDtypeStruct((_K,), jnp.int32),
            jax.ShapeDtypeStruct((_K,), jnp.int32),
            jax.ShapeDtypeStruct((_K,), jnp.int32),
            jax.ShapeDtypeStruct((_K,), jnp.float32),
        ),
    )(gumbel_kk, log_weight_k, scalar_tbl)


def _sc_gather_body(
    it0_hbm, it1_hbm, lf_hbm, emb_hbm,
    lf0_out, lf1_out, emb_out,
    it0_v, it1_v, lf0_v, lf1_v, e0_v, e1_v, es_v,
    sem0, sem1,
):
    w = lax.axis_index("s") * _NC + lax.axis_index("c")
    base = w * _PPW

    pltpu.sync_copy(it0_hbm.at[pl.ds(base, _PPW)], it0_v)
    pltpu.sync_copy(it1_hbm.at[pl.ds(base, _PPW)], it1_v)

    cp0 = pltpu.async_copy(lf_hbm.at[it0_v], lf0_v, sem0)
    cp1 = pltpu.async_copy(lf_hbm.at[it1_v], lf1_v, sem0)
    ce0 = pltpu.async_copy(emb_hbm.at[it0_v], e0_v, sem1)
    ce1 = pltpu.async_copy(emb_hbm.at[it1_v], e1_v, sem1)
    ce0.wait()
    ce1.wait()
    for p in range(_PPW):
        for jc in range(_D // 16):
            sl = pl.ds(jc * 16, 16)
            es_v[p, sl] = (e0_v[p, sl] + e1_v[p, sl]) * 0.5
    pltpu.sync_copy(es_v, emb_out.at[pl.ds(base, _PPW)])
    cp0.wait()
    cp1.wait()
    pltpu.sync_copy(lf0_v, lf0_out.at[pl.ds(base, _PPW)])
    pltpu.sync_copy(lf1_v, lf1_out.at[pl.ds(base, _PPW)])


def _sc_gather(it0, it1, lf01p, emb01):
    mesh = plsc.VectorSubcoreMesh(
        core_axis_name="c", subcore_axis_name="s", num_cores=_NC,
        num_subcores=_NS)
    f = pl.kernel(
        _sc_gather_body,
        out_type=(
            jax.ShapeDtypeStruct((_K, _SAP), jnp.float32),
            jax.ShapeDtypeStruct((_K, _SAP), jnp.float32),
            jax.ShapeDtypeStruct((_K, _D), jnp.float32),
        ),
        mesh=mesh,
        scratch_types=[
            pltpu.VMEM((_PPW,), jnp.int32),
            pltpu.VMEM((_PPW,), jnp.int32),
            pltpu.VMEM((_PPW, _SAP), jnp.float32),
            pltpu.VMEM((_PPW, _SAP), jnp.float32),
            pltpu.VMEM((_PPW, _D), jnp.float32),
            pltpu.VMEM((_PPW, _D), jnp.float32),
            pltpu.VMEM((_PPW, _D), jnp.float32),
            pltpu.SemaphoreType.DMA,
            pltpu.SemaphoreType.DMA,
        ],
    )
    return f(it0, it1, lf01p, emb01)


def _grp4(x):
    lane = lax.broadcasted_iota(jnp.int32, x.shape, 1)
    p2 = x + jnp.where(lane & 1 == 1, jnp.roll(x, 1, axis=1),
                       jnp.roll(x, -1, axis=1))
    return p2 + jnp.where(lane & 2 == 2, jnp.roll(p2, 2, axis=1),
                          jnp.roll(p2, -2, axis=1))


def _merge_body(lf0_ref, lf1_ref, lp_ref, lfn_ref, lw_ref):
    eb = jnp.exp(jnp.float32(-0.1))
    move = (1.0 - eb) / _A
    e1 = jnp.exp(lf0_ref[...])
    e2 = jnp.exp(lf1_ref[...])
    q1 = move * _grp4(e1) + eb * e1
    q2 = move * _grp4(e2) + eb * e2
    u = q1 * q2
    lfn_ref[...] = jnp.log(u)
    lane = lax.broadcasted_iota(jnp.int32, u.shape, 1)
    tu = jnp.where(lane < _SA, _grp4(u), 1.0)
    ll = 0.25 * jnp.sum(jnp.log(tu), axis=1) - _S * math.log(_A)
    lw_ref[...] = (ll - lp_ref[0, 0, :])[None, None, :]


_BK = 128


def _merge(lf0, lf1, lp_g):
    grid = (_K // _BK,)
    lf_new2d, lw2d = pl.pallas_call(
        _merge_body,
        grid=grid,
        in_specs=[
            pl.BlockSpec((_BK, _SAP), lambda i: (i, 0)),
            pl.BlockSpec((_BK, _SAP), lambda i: (i, 0)),
            pl.BlockSpec((1, 1, _BK), lambda i: (i, 0, 0)),
        ],
        out_specs=(
            pl.BlockSpec((_BK, _SAP), lambda i: (i, 0)),
            pl.BlockSpec((1, 1, _BK), lambda i: (i, 0, 0)),
        ),
        out_shape=(
            jax.ShapeDtypeStruct((_K, _SAP), jnp.float32),
            jax.ShapeDtypeStruct((_K // _BK, 1, _BK), jnp.float32),
        ),
    )(lf0, lf1, lp_g.reshape(_K // _BK, 1, _BK))
    return lf_new2d, lw2d.reshape(_K)


def kernel(log_weight_K, log_pi_K, log_felsensteins_KxtxSxA, embeddings_KxtxD,
           leaf_counts_Kxt, hashes_Kxt):
    gumbel = jax.random.gumbel(jax.random.key(1), (_K, _K), jnp.float32)

    lf01p = jnp.pad(
        log_felsensteins_KxtxSxA[:, :2].reshape(_K * 2, _SA),
        ((0, 0), (0, _SAP - _SA)))
    emb01 = embeddings_KxtxD[:, :2].reshape(_K * 2, _D)
    scalar_tbl = jnp.stack(
        [leaf_counts_Kxt[:, 0], leaf_counts_Kxt[:, 1],
         hashes_Kxt[:, 0], hashes_Kxt[:, 1],
         lax.bitcast_convert_type(log_pi_K, jnp.int32)], axis=0)

    it0, it1, lc_new, hs_new, lp_g = _sample(gumbel, log_weight_K, scalar_tbl)
    lf0, lf1, emb_new = _sc_gather(it0, it1, lf01p, emb01)
    lf_new2d, lw_new = _merge(lf0, lf1, lp_g)
    lf_new = lf_new2d[:, :_SA].reshape(_K, _S, _A)
    return (lw_new, lf_new, emb_new, lc_new, hs_new)


# speedup vs baseline: 14.7538x; 4.6450x over previous
import math

import jax
import jax.numpy as jnp
from jax import lax
from jax.experimental import pallas as pl
from jax.experimental.pallas import tpu as pltpu
from jax.experimental.pallas import tpu_sc as plsc

_K = 256
_T = 64
_S = 500
_A = 4
_D = 128

_NC = 2
_NS = 16
_NW = _NC * _NS
_PPW = _K // _NW


def _sample_body(g_ref, lw_ref, sc_ref, it0_ref, it1_ref, oh_ref, lc_ref,
                 hs_ref, lp_ref):
    x = g_ref[...] + lw_ref[...][None, :]
    m = jnp.max(x, axis=1, keepdims=True)
    ii = lax.broadcasted_iota(jnp.int32, x.shape, 1)
    cand = jnp.where(x >= m, ii, jnp.int32(x.shape[1]))
    idx = jnp.min(cand, axis=1)
    it0_ref[...] = idx * 2
    it1_ref[...] = idx * 2 + 1
    eq = idx[:, None] == ii
    oh_ref[...] = eq.astype(jnp.float32)
    t = sc_ref[...]

    def pick(row):
        return jnp.sum(jnp.where(eq, row[None, :], 0), axis=1)

    lc_ref[...] = pick(t[0]) + pick(t[1])
    hs_ref[...] = pick(t[2]) * jnp.int32(1000003) + pick(t[3])
    lp_ref[...] = lax.bitcast_convert_type(pick(t[4]), jnp.float32)


def _sample(gumbel_kk, log_weight_k, scalar_tbl):
    return pl.pallas_call(
        _sample_body,
        out_shape=(
            jax.ShapeDtypeStruct((_K,), jnp.int32),
            jax.ShapeDtypeStruct((_K,), jnp.int32),
            jax.ShapeDtypeStruct((_K, _K), jnp.float32),
            jax.ShapeDtypeStruct((_K,), jnp.int32),
            jax.ShapeDtypeStruct((_K,), jnp.int32),
            jax.ShapeDtypeStruct((_K,), jnp.float32),
        ),
    )(gumbel_kk, log_weight_k, scalar_tbl)


def _sc_gather_body(
    it0_hbm, it1_hbm, emb_hbm,
    emb_out,
    it0_v, it1_v, e0_v, e1_v, es_v,
    sem0,
):
    w = lax.axis_index("s") * _NC + lax.axis_index("c")
    base = w * _PPW

    pltpu.sync_copy(it0_hbm.at[pl.ds(base, _PPW)], it0_v)
    pltpu.sync_copy(it1_hbm.at[pl.ds(base, _PPW)], it1_v)
    ce0 = pltpu.async_copy(emb_hbm.at[it0_v], e0_v, sem0)
    ce1 = pltpu.async_copy(emb_hbm.at[it1_v], e1_v, sem0)
    ce0.wait()
    ce1.wait()
    for p in range(_PPW):
        for jc in range(_D // 16):
            sl = pl.ds(jc * 16, 16)
            es_v[p, sl] = (e0_v[p, sl] + e1_v[p, sl]) * 0.5
    pltpu.sync_copy(es_v, emb_out.at[pl.ds(base, _PPW)])


def _sc_gather(it0, it1, emb01):
    mesh = plsc.VectorSubcoreMesh(
        core_axis_name="c", subcore_axis_name="s", num_cores=_NC,
        num_subcores=_NS)
    f = pl.kernel(
        _sc_gather_body,
        out_type=jax.ShapeDtypeStruct((_K, _D), jnp.float32),
        mesh=mesh,
        scratch_types=[
            pltpu.VMEM((_PPW,), jnp.int32),
            pltpu.VMEM((_PPW,), jnp.int32),
            pltpu.VMEM((_PPW, _D), jnp.float32),
            pltpu.VMEM((_PPW, _D), jnp.float32),
            pltpu.VMEM((_PPW, _D), jnp.float32),
            pltpu.SemaphoreType.DMA,
        ],
    )
    return f(it0, it1, emb01)


_DN_LFN = (((2,), (1,)), ((), ()))
_DN_LL = (((1,), (1,)), ((), ()))


def _merge_body(l0_ref, l1_ref, oh_ref, lp_ref, lfn_ref, lw_ref):
    eb = jnp.exp(jnp.float32(-0.1))
    move = (1.0 - eb) / _A
    e1 = jnp.exp(l0_ref[...])
    e2 = jnp.exp(l1_ref[...])
    q1 = move * jnp.sum(e1, axis=1, keepdims=True) + eb * e1
    q2 = move * jnp.sum(e2, axis=1, keepdims=True) + eb * e2
    u = q1 * q2
    lfn_j = jnp.log(u)
    ll_j = (jnp.sum(jnp.log(jnp.sum(u, axis=1)), axis=0)
            - _S * math.log(_A))
    oh = oh_ref[...]
    lfn_ref[...] = lax.dot_general(
        lfn_j, oh, _DN_LFN, precision=lax.Precision.HIGHEST,
        preferred_element_type=jnp.float32)
    ll = lax.dot_general(
        ll_j[None, :], oh, _DN_LL, precision=lax.Precision.HIGHEST,
        preferred_element_type=jnp.float32)[0]
    lw_ref[...] = ll - lp_ref[...]


def _merge(lft0, lft1, ohf, lp_g):
    return pl.pallas_call(
        _merge_body,
        out_shape=(
            jax.ShapeDtypeStruct((_S, _A, _K), jnp.float32),
            jax.ShapeDtypeStruct((_K,), jnp.float32),
        ),
    )(lft0, lft1, ohf, lp_g)


def kernel(log_weight_K, log_pi_K, log_felsensteins_KxtxSxA, embeddings_KxtxD,
           leaf_counts_Kxt, hashes_Kxt):
    gumbel = jax.random.gumbel(jax.random.key(1), (_K, _K), jnp.float32)

    scalar_tbl = jnp.stack(
        [leaf_counts_Kxt[:, 0], leaf_counts_Kxt[:, 1],
         hashes_Kxt[:, 0], hashes_Kxt[:, 1],
         lax.bitcast_convert_type(log_pi_K, jnp.int32)], axis=0)
    emb01 = embeddings_KxtxD[:, :2].reshape(_K * 2, _D)

    it0, it1, ohf, lc_new, hs_new, lp_g = _sample(
        gumbel, log_weight_K, scalar_tbl)
    emb_new = _sc_gather(it0, it1, emb01)

    lf_t = jnp.transpose(log_felsensteins_KxtxSxA, (1, 2, 3, 0))
    lfn_t, lw_new = _merge(lf_t[0], lf_t[1], ohf, lp_g)
    lf_new = jnp.transpose(lfn_t, (2, 0, 1))
    return (lw_new, lf_new, emb_new, lc_new, hs_new)


# speedup vs baseline: 17.9222x; 1.2147x over previous
import math

import jax
import jax.numpy as jnp
from jax import lax
from jax.experimental import pallas as pl
from jax.experimental.pallas import tpu as pltpu
from jax.experimental.pallas import tpu_sc as plsc

_K = 256
_T = 64
_S = 500
_A = 4
_D = 128

_NC = 2
_NS = 16
_NW = _NC * _NS
_PPW = _K // _NW


def _sample_body(g_ref, lw_ref, sc_ref, it0_ref, it1_ref, oh_ref, lc_ref,
                 hs_ref, lp_ref):
    x = g_ref[...] + lw_ref[...][None, :]
    m = jnp.max(x, axis=1, keepdims=True)
    ii = lax.broadcasted_iota(jnp.int32, x.shape, 1)
    cand = jnp.where(x >= m, ii, jnp.int32(x.shape[1]))
    idx = jnp.min(cand, axis=1)
    it0_ref[...] = idx * _T
    it1_ref[...] = idx * _T + 1
    eq = idx[:, None] == ii
    oh_ref[...] = eq.astype(jnp.float32)
    t = sc_ref[...]

    def pick(row):
        return jnp.sum(jnp.where(eq, row[None, :], 0), axis=1)

    lc_ref[...] = pick(t[0]) + pick(t[1])
    hs_ref[...] = pick(t[2]) * jnp.int32(1000003) + pick(t[3])
    lp_ref[...] = lax.bitcast_convert_type(pick(t[4]), jnp.float32)


def _sample(gumbel_kk, log_weight_k, scalar_tbl):
    return pl.pallas_call(
        _sample_body,
        out_shape=(
            jax.ShapeDtypeStruct((_K,), jnp.int32),
            jax.ShapeDtypeStruct((_K,), jnp.int32),
            jax.ShapeDtypeStruct((_K, _K), jnp.float32),
            jax.ShapeDtypeStruct((_K,), jnp.int32),
            jax.ShapeDtypeStruct((_K,), jnp.int32),
            jax.ShapeDtypeStruct((_K,), jnp.float32),
        ),
    )(gumbel_kk, log_weight_k, scalar_tbl)


def _sc_gather_body(
    it0_hbm, it1_hbm, emb_hbm,
    emb_out,
    it0_v, it1_v, e0_v, e1_v, es_v,
    sem0,
):
    w = lax.axis_index("s") * _NC + lax.axis_index("c")
    base = w * _PPW

    pltpu.sync_copy(it0_hbm.at[pl.ds(base, _PPW)], it0_v)
    pltpu.sync_copy(it1_hbm.at[pl.ds(base, _PPW)], it1_v)
    ce0 = pltpu.async_copy(emb_hbm.at[it0_v], e0_v, sem0)
    ce1 = pltpu.async_copy(emb_hbm.at[it1_v], e1_v, sem0)
    ce0.wait()
    ce1.wait()
    for p in range(_PPW):
        for jc in range(_D // 16):
            sl = pl.ds(jc * 16, 16)
            es_v[p, sl] = (e0_v[p, sl] + e1_v[p, sl]) * 0.5
    pltpu.sync_copy(es_v, emb_out.at[pl.ds(base, _PPW)])


def _sc_gather(it0, it1, emb01):
    mesh = plsc.VectorSubcoreMesh(
        core_axis_name="c", subcore_axis_name="s", num_cores=_NC,
        num_subcores=_NS)
    f = pl.kernel(
        _sc_gather_body,
        out_type=jax.ShapeDtypeStruct((_K, _D), jnp.float32),
        mesh=mesh,
        scratch_types=[
            pltpu.VMEM((_PPW,), jnp.int32),
            pltpu.VMEM((_PPW,), jnp.int32),
            pltpu.VMEM((_PPW, _D), jnp.float32),
            pltpu.VMEM((_PPW, _D), jnp.float32),
            pltpu.VMEM((_PPW, _D), jnp.float32),
            pltpu.SemaphoreType.DMA,
        ],
    )
    return f(it0, it1, emb01)


_DN_LFN = (((2,), (1,)), ((), ()))
_DN_LL = (((1,), (1,)), ((), ()))
_PREC = lax.Precision.HIGHEST
_GS = 4
_BS = _S // _GS


def _merge_body(l0_ref, l1_ref, oh_ref, lp_ref, lfn_ref, lw_ref):
    i = pl.program_id(0)
    eb = jnp.exp(jnp.float32(-0.1))
    move = (1.0 - eb) / _A
    e1 = jnp.exp(l0_ref[0])
    e2 = jnp.exp(l1_ref[0])
    q1 = move * jnp.sum(e1, axis=1, keepdims=True) + eb * e1
    q2 = move * jnp.sum(e2, axis=1, keepdims=True) + eb * e2
    u = q1 * q2
    oh = oh_ref[...]
    lfn_ref[...] = lax.dot_general(
        jnp.log(u), oh, _DN_LFN, precision=_PREC,
        preferred_element_type=jnp.float32)
    part = jnp.sum(jnp.log(jnp.sum(u, axis=1)), axis=0)

    @pl.when(i == 0)
    def _init():
        lw_ref[...] = part

    @pl.when(jnp.logical_and(i > 0, i < _GS - 1))
    def _acc():
        lw_ref[...] = lw_ref[...] + part

    @pl.when(i == _GS - 1)
    def _fin():
        ll_j = lw_ref[...] + part - _S * math.log(_A)
        ll = lax.dot_general(ll_j[None, :], oh, _DN_LL, precision=_PREC,
                             preferred_element_type=jnp.float32)[0]
        lw_ref[...] = ll - lp_ref[...]


def _merge(lf_t, ohf, lp_g):
    return pl.pallas_call(
        _merge_body,
        grid=(_GS,),
        in_specs=[
            pl.BlockSpec((1, _BS, _A, _K), lambda i: (0, i, 0, 0)),
            pl.BlockSpec((1, _BS, _A, _K), lambda i: (1, i, 0, 0)),
            pl.BlockSpec((_K, _K), lambda i: (0, 0)),
            pl.BlockSpec((_K,), lambda i: (0,)),
        ],
        out_specs=(
            pl.BlockSpec((_BS, _A, _K), lambda i: (i, 0, 0)),
            pl.BlockSpec((_K,), lambda i: (0,)),
        ),
        out_shape=(
            jax.ShapeDtypeStruct((_S, _A, _K), jnp.float32),
            jax.ShapeDtypeStruct((_K,), jnp.float32),
        ),
    )(lf_t, lf_t, ohf, lp_g)


def kernel(log_weight_K, log_pi_K, log_felsensteins_KxtxSxA, embeddings_KxtxD,
           leaf_counts_Kxt, hashes_Kxt):
    gumbel = jax.random.gumbel(jax.random.key(1), (_K, _K), jnp.float32)

    scalar_tbl = jnp.stack(
        [leaf_counts_Kxt[:, 0], leaf_counts_Kxt[:, 1],
         hashes_Kxt[:, 0], hashes_Kxt[:, 1],
         lax.bitcast_convert_type(log_pi_K, jnp.int32)], axis=0)
    emb_flat = embeddings_KxtxD.reshape(_K * _T, _D)

    it0, it1, ohf, lc_new, hs_new, lp_g = _sample(
        gumbel, log_weight_K, scalar_tbl)
    emb_new = _sc_gather(it0, it1, emb_flat)

    lf_t = jnp.transpose(log_felsensteins_KxtxSxA, (1, 2, 3, 0))
    lfn_t, lw_new = _merge(lf_t, ohf, lp_g)
    lf_new = jnp.transpose(lfn_t, (2, 0, 1))
    return (lw_new, lf_new, emb_new, lc_new, hs_new)
